# Initial kernel scaffold; baseline (speedup 1.0000x reference)
import jax
import jax.numpy as jnp
from jax.experimental import pallas as pl


def kernel(input, head_w, tail0_emb, tail0_lin, tail1_emb, tail1_lin):
    raise NotImplementedError("write your pallas kernel here")


# speedup vs baseline: 25.5550x; 25.5550x over previous
import functools

import jax
import jax.numpy as jnp
from jax import lax
from jax.experimental import pallas as pl
from jax.experimental.pallas import tpu as pltpu
from jax.experimental.pallas import tpu_sc as plsc

_DIM = 128
_HEAD = 5000
_CUT1 = 25000
_VOCAB = 100000
_ROWS_BLK = 1000


def _table_body(head_ref, t0e_ref, t0l_ref, t1e_ref, t1l_ref, out_ref):
    i = pl.program_id(0)

    @pl.when(i < _HEAD // _ROWS_BLK)
    def _():
        out_ref[...] = head_ref[...]

    @pl.when(jnp.logical_and(i >= _HEAD // _ROWS_BLK, i < _CUT1 // _ROWS_BLK))
    def _():
        out_ref[...] = lax.dot_general(
            t0e_ref[...], t0l_ref[...],
            (((1,), (1,)), ((), ())),
            preferred_element_type=jnp.float32,
        )

    @pl.when(i >= _CUT1 // _ROWS_BLK)
    def _():
        out_ref[...] = lax.dot_general(
            t1e_ref[...], t1l_ref[...],
            (((1,), (1,)), ((), ())),
            preferred_element_type=jnp.float32,
        )


def _build_table(head_w, tail0_emb, tail0_lin, tail1_emb, tail1_lin):
    nblk = _VOCAB // _ROWS_BLK
    h_blk = _HEAD // _ROWS_BLK
    c1_blk = _CUT1 // _ROWS_BLK
    return pl.pallas_call(
        _table_body,
        grid=(nblk,),
        in_specs=[
            pl.BlockSpec((_ROWS_BLK, _DIM),
                         lambda i: (jnp.minimum(i, h_blk - 1), 0)),
            pl.BlockSpec((_ROWS_BLK, 64),
                         lambda i: (jnp.clip(i - h_blk, 0, c1_blk - h_blk - 1), 0)),
            pl.BlockSpec((_DIM, 64), lambda i: (0, 0)),
            pl.BlockSpec((_ROWS_BLK, 32),
                         lambda i: (jnp.clip(i - c1_blk, 0, nblk - c1_blk - 1), 0)),
            pl.BlockSpec((_DIM, 32), lambda i: (0, 0)),
        ],
        out_specs=pl.BlockSpec((_ROWS_BLK, _DIM), lambda i: (i, 0)),
        out_shape=jax.ShapeDtypeStruct((_VOCAB, _DIM), jnp.float32),
    )(head_w, tail0_emb, tail0_lin, tail1_emb, tail1_lin)


def _make_gather(n_tokens):
    info = plsc.get_sparse_core_info()
    nc, ns = info.num_cores, info.num_subcores
    nw = nc * ns
    chunk = 128
    nbuf = 4
    per_w = n_tokens // nw
    n_chunks = per_w // chunk
    assert per_w % chunk == 0 and n_chunks % nbuf == 0
    mesh = plsc.VectorSubcoreMesh(core_axis_name="c", subcore_axis_name="s")

    @functools.partial(
        pl.kernel, mesh=mesh,
        out_type=jax.ShapeDtypeStruct((n_tokens, _DIM), jnp.float32),
        scratch_types=[
            pltpu.VMEM((n_chunks, chunk), jnp.int32),
            pltpu.VMEM((chunk, _DIM), jnp.float32),
            pltpu.VMEM((chunk, _DIM), jnp.float32),
            pltpu.VMEM((chunk, _DIM), jnp.float32),
            pltpu.VMEM((chunk, _DIM), jnp.float32),
            pltpu.SemaphoreType.DMA,
            pltpu.SemaphoreType.DMA,
            pltpu.SemaphoreType.DMA,
            pltpu.SemaphoreType.DMA,
        ],
    )
    def gather(table_hbm, idx_hbm, out_hbm,
               idx_v, b0, b1, b2, b3, s0, s1, s2, s3):
        bufs = (b0, b1, b2, b3)
        sems = (s0, s1, s2, s3)
        wid = lax.axis_index("s") * nc + lax.axis_index("c")
        row0 = wid * per_w
        pltpu.sync_copy(idx_hbm.at[pl.ds(wid * n_chunks, n_chunks)], idx_v)
        for b in range(nbuf):
            pltpu.async_copy(table_hbm.at[idx_v.at[b]], bufs[b], sems[b])

        def body(g, _):
            for b in range(nbuf):
                j = g * nbuf + b
                pltpu.make_async_copy(
                    table_hbm.at[idx_v.at[j]], bufs[b], sems[b]).wait()
                pltpu.sync_copy(
                    bufs[b], out_hbm.at[pl.ds(row0 + j * chunk, chunk)])

                @pl.when(j + nbuf < n_chunks)
                def _():
                    pltpu.async_copy(
                        table_hbm.at[idx_v.at[j + nbuf]], bufs[b], sems[b])

        lax.fori_loop(0, n_chunks // nbuf, body, None)

    return gather


def kernel(input, head_w, tail0_emb, tail0_lin, tail1_emb, tail1_lin):
    B, L = input.shape
    n_tokens = B * L
    table = _build_table(head_w, tail0_emb, tail0_lin, tail1_emb, tail1_lin)
    idx2d = input.reshape(n_tokens // 128, 128)
    out = _make_gather(n_tokens)(table, idx2d)
    return out.reshape(B, L, _DIM)


# speedup vs baseline: 28.2442x; 1.1052x over previous
import functools

import jax
import jax.numpy as jnp
from jax import lax
from jax.experimental import pallas as pl
from jax.experimental.pallas import tpu as pltpu
from jax.experimental.pallas import tpu_sc as plsc

_DIM = 128
_HEAD = 5000
_CUT1 = 25000
_VOCAB = 100000
_ROWS_BLK = 5000


def _table_body(head_ref, t0e_ref, t0l_ref, t1e_ref, t1l_ref, out_ref):
    i = pl.program_id(0)

    @pl.when(i < _HEAD // _ROWS_BLK)
    def _():
        out_ref[...] = head_ref[...]

    @pl.when(jnp.logical_and(i >= _HEAD // _ROWS_BLK, i < _CUT1 // _ROWS_BLK))
    def _():
        out_ref[...] = lax.dot_general(
            t0e_ref[...], t0l_ref[...],
            (((1,), (1,)), ((), ())),
            preferred_element_type=jnp.float32,
        )

    @pl.when(i >= _CUT1 // _ROWS_BLK)
    def _():
        out_ref[...] = lax.dot_general(
            t1e_ref[...], t1l_ref[...],
            (((1,), (1,)), ((), ())),
            preferred_element_type=jnp.float32,
        )


def _build_table(head_w, tail0_emb, tail0_lin, tail1_emb, tail1_lin):
    nblk = _VOCAB // _ROWS_BLK
    h_blk = _HEAD // _ROWS_BLK
    c1_blk = _CUT1 // _ROWS_BLK
    return pl.pallas_call(
        _table_body,
        grid=(nblk,),
        in_specs=[
            pl.BlockSpec((_ROWS_BLK, _DIM),
                         lambda i: (jnp.minimum(i, h_blk - 1), 0)),
            pl.BlockSpec((_ROWS_BLK, 64),
                         lambda i: (jnp.clip(i - h_blk, 0, c1_blk - h_blk - 1), 0)),
            pl.BlockSpec((_DIM, 64), lambda i: (0, 0)),
            pl.BlockSpec((_ROWS_BLK, 32),
                         lambda i: (jnp.clip(i - c1_blk, 0, nblk - c1_blk - 1), 0)),
            pl.BlockSpec((_DIM, 32), lambda i: (0, 0)),
        ],
        out_specs=pl.BlockSpec((_ROWS_BLK, _DIM), lambda i: (i, 0)),
        out_shape=jax.ShapeDtypeStruct((_VOCAB, _DIM), jnp.float32),
    )(head_w, tail0_emb, tail0_lin, tail1_emb, tail1_lin)


def _make_gather(n_tokens):
    info = plsc.get_sparse_core_info()
    nc, ns = info.num_cores, info.num_subcores
    nw = nc * ns
    chunk = 128
    nbuf = 5
    per_w = n_tokens // nw
    n_chunks = per_w // chunk
    assert per_w % chunk == 0 and n_chunks % nbuf == 0
    mesh = plsc.VectorSubcoreMesh(core_axis_name="c", subcore_axis_name="s")

    @functools.partial(
        pl.kernel, mesh=mesh,
        out_type=jax.ShapeDtypeStruct((n_tokens, _DIM), jnp.float32),
        scratch_types=[
            pltpu.VMEM((n_chunks, chunk), jnp.int32),
        ]
        + [pltpu.VMEM((chunk, _DIM), jnp.float32)] * nbuf
        + [pltpu.SemaphoreType.DMA] * (2 * nbuf),
    )
    def gather(table_hbm, idx_hbm, out_hbm, idx_v, *bufsem):
        bufs = bufsem[:nbuf]
        gsem = bufsem[nbuf:2 * nbuf]
        ssem = bufsem[2 * nbuf:]
        wid = lax.axis_index("s") * nc + lax.axis_index("c")
        row0 = wid * per_w
        pltpu.sync_copy(idx_hbm.at[pl.ds(wid * n_chunks, n_chunks)], idx_v)
        for b in range(nbuf - 1):
            pltpu.async_copy(table_hbm.at[idx_v.at[b]], bufs[b], gsem[b])

        def body(g, _):
            for b in range(nbuf):
                j = g * nbuf + b
                pltpu.make_async_copy(
                    table_hbm.at[idx_v.at[j]], bufs[b], gsem[b]).wait()
                pltpu.async_copy(
                    bufs[b], out_hbm.at[pl.ds(row0 + j * chunk, chunk)],
                    ssem[b])
                jn = j + nbuf - 1
                bn = (b - 1) % nbuf

                @pl.when(jn < n_chunks)
                def _():
                    @pl.when(j >= 1)
                    def _():
                        pltpu.make_async_copy(
                            bufs[bn],
                            out_hbm.at[pl.ds(row0, chunk)],
                            ssem[bn]).wait()

                    pltpu.async_copy(
                        table_hbm.at[idx_v.at[jn]], bufs[bn], gsem[bn])

        lax.fori_loop(0, n_chunks // nbuf, body, None)
        for b in range(nbuf):
            pltpu.make_async_copy(
                bufs[b], out_hbm.at[pl.ds(row0, chunk)], ssem[b]).wait()

    return gather


def kernel(input, head_w, tail0_emb, tail0_lin, tail1_emb, tail1_lin):
    B, L = input.shape
    n_tokens = B * L
    table = _build_table(head_w, tail0_emb, tail0_lin, tail1_emb, tail1_lin)
    idx2d = input.reshape(n_tokens // 128, 128)
    out = _make_gather(n_tokens)(table, idx2d)
    return out.reshape(B, L, _DIM)


# speedup vs baseline: 28.3318x; 1.0031x over previous
import functools

import jax
import jax.numpy as jnp
from jax import lax
from jax.experimental import pallas as pl
from jax.experimental.pallas import tpu as pltpu
from jax.experimental.pallas import tpu_sc as plsc

_DIM = 128
_HEAD = 5000
_CUT1 = 25000
_VOCAB = 100000
_ROWS_BLK = 5000


def _table_body(head_ref, t0e_ref, t0l_ref, t1e_ref, t1l_ref, out_ref):
    i = pl.program_id(0)

    @pl.when(i < _HEAD // _ROWS_BLK)
    def _():
        out_ref[...] = head_ref[...]

    @pl.when(jnp.logical_and(i >= _HEAD // _ROWS_BLK, i < _CUT1 // _ROWS_BLK))
    def _():
        out_ref[...] = lax.dot_general(
            t0e_ref[...], t0l_ref[...],
            (((1,), (1,)), ((), ())),
            preferred_element_type=jnp.float32,
        )

    @pl.when(i >= _CUT1 // _ROWS_BLK)
    def _():
        out_ref[...] = lax.dot_general(
            t1e_ref[...], t1l_ref[...],
            (((1,), (1,)), ((), ())),
            preferred_element_type=jnp.float32,
        )


def _build_table(head_w, tail0_emb, tail0_lin, tail1_emb, tail1_lin):
    nblk = _VOCAB // _ROWS_BLK
    h_blk = _HEAD // _ROWS_BLK
    c1_blk = _CUT1 // _ROWS_BLK
    return pl.pallas_call(
        _table_body,
        grid=(nblk,),
        in_specs=[
            pl.BlockSpec((_ROWS_BLK, _DIM),
                         lambda i: (jnp.minimum(i, h_blk - 1), 0)),
            pl.BlockSpec((_ROWS_BLK, 64),
                         lambda i: (jnp.clip(i - h_blk, 0, c1_blk - h_blk - 1), 0)),
            pl.BlockSpec((_DIM, 64), lambda i: (0, 0)),
            pl.BlockSpec((_ROWS_BLK, 32),
                         lambda i: (jnp.clip(i - c1_blk, 0, nblk - c1_blk - 1), 0)),
            pl.BlockSpec((_DIM, 32), lambda i: (0, 0)),
        ],
        out_specs=pl.BlockSpec((_ROWS_BLK, _DIM), lambda i: (i, 0)),
        out_shape=jax.ShapeDtypeStruct((_VOCAB, _DIM), jnp.float32),
    )(head_w, tail0_emb, tail0_lin, tail1_emb, tail1_lin)


def _make_gather(n_tokens):
    info = plsc.get_sparse_core_info()
    nc, ns = info.num_cores, info.num_subcores
    nw = nc * ns
    chunk = 128
    spc = 2
    slot_rows = chunk * spc
    nbuf = 3
    per_w = n_tokens // nw
    n_chunks = per_w // chunk
    n_slots = n_chunks // spc
    assert per_w % slot_rows == 0 and n_slots % nbuf == 1
    mesh = plsc.VectorSubcoreMesh(core_axis_name="c", subcore_axis_name="s")

    @functools.partial(
        pl.kernel, mesh=mesh,
        out_type=jax.ShapeDtypeStruct((n_tokens, _DIM), jnp.float32),
        scratch_types=[
            pltpu.VMEM((n_chunks, chunk), jnp.int32),
        ]
        + [pltpu.VMEM((slot_rows, _DIM), jnp.float32)] * nbuf
        + [pltpu.SemaphoreType.DMA] * (2 * nbuf),
    )
    def gather(table_hbm, idx_hbm, out_hbm, idx_v, *bufsem):
        bufs = bufsem[:nbuf]
        gsem = bufsem[nbuf:2 * nbuf]
        ssem = bufsem[2 * nbuf:]
        wid = lax.axis_index("s") * nc + lax.axis_index("c")
        row0 = wid * per_w
        pltpu.sync_copy(idx_hbm.at[pl.ds(wid * n_chunks, n_chunks)], idx_v)

        def start_slot(s, b):
            for c in range(spc):
                pltpu.async_copy(
                    table_hbm.at[idx_v.at[s * spc + c]],
                    bufs[b].at[pl.ds(c * chunk, chunk)], gsem[b])

        def wait_slot(s, b):
            for c in range(spc):
                pltpu.make_async_copy(
                    table_hbm.at[idx_v.at[s * spc + c]],
                    bufs[b].at[pl.ds(c * chunk, chunk)], gsem[b]).wait()

        def wait_store(b):
            pltpu.make_async_copy(
                bufs[b], out_hbm.at[pl.ds(row0, slot_rows)], ssem[b]).wait()

        for b in range(nbuf - 1):
            start_slot(b, b)

        def body(g, _):
            for b in range(nbuf):
                s = g * nbuf + b
                wait_slot(s, b)
                pltpu.async_copy(
                    bufs[b], out_hbm.at[pl.ds(row0 + s * slot_rows, slot_rows)],
                    ssem[b])
                sn = s + nbuf - 1
                bn = (b - 1) % nbuf

                @pl.when(sn < n_slots)
                def _():
                    @pl.when(s >= 1)
                    def _():
                        wait_store(bn)

                    start_slot(sn, bn)

        lax.fori_loop(0, n_slots // nbuf, body, None)
        s_last = n_slots - 1
        b_last = s_last % nbuf
        wait_slot(s_last, b_last)
        pltpu.async_copy(
            bufs[b_last],
            out_hbm.at[pl.ds(row0 + s_last * slot_rows, slot_rows)],
            ssem[b_last])
        for b in range(nbuf):
            wait_store(b)

    return gather


def kernel(input, head_w, tail0_emb, tail0_lin, tail1_emb, tail1_lin):
    B, L = input.shape
    n_tokens = B * L
    table = _build_table(head_w, tail0_emb, tail0_lin, tail1_emb, tail1_lin)
    idx2d = input.reshape(n_tokens // 128, 128)
    out = _make_gather(n_tokens)(table, idx2d)
    return out.reshape(B, L, _DIM)


# speedup vs baseline: 28.4249x; 1.0033x over previous
import functools

import jax
import jax.numpy as jnp
from jax import lax
from jax.experimental import pallas as pl
from jax.experimental.pallas import tpu as pltpu
from jax.experimental.pallas import tpu_sc as plsc

_DIM = 128
_HEAD = 5000
_CUT1 = 25000
_VOCAB = 100000
_ROWS_BLK = 5000


def _head_body(head_ref, out_ref):
    out_ref[...] = head_ref[...]


def _proj_body(emb_ref, lin_ref, tbl_ref, out_ref):
    del tbl_ref
    out_ref[...] = lax.dot_general(
        emb_ref[...], lin_ref[...],
        (((1,), (1,)), ((), ())),
        preferred_element_type=jnp.float32,
    )


def _build_table(head_w, tail0_emb, tail0_lin, tail1_emb, tail1_lin):
    tshape = jax.ShapeDtypeStruct((_VOCAB, _DIM), jnp.float32)
    nb = _ROWS_BLK
    tbl = pl.pallas_call(
        _head_body,
        grid=(_HEAD // nb,),
        in_specs=[pl.BlockSpec((nb, _DIM), lambda i: (i, 0))],
        out_specs=pl.BlockSpec((nb, _DIM), lambda i: (i, 0)),
        out_shape=tshape,
    )(head_w)

    def proj(tbl, emb, lin, row_off, n_rows, hsz, pb):
        blk0 = row_off // pb
        assert row_off % pb == 0 and n_rows % pb == 0
        return pl.pallas_call(
            _proj_body,
            grid=(n_rows // pb,),
            in_specs=[
                pl.BlockSpec((pb, hsz), lambda i: (i, 0)),
                pl.BlockSpec((_DIM, hsz), lambda i: (0, 0)),
                pl.BlockSpec(memory_space=pl.ANY),
            ],
            out_specs=pl.BlockSpec((pb, _DIM), lambda i: (i + blk0, 0)),
            out_shape=tshape,
            input_output_aliases={2: 0},
        )(emb, lin, tbl)

    tbl = proj(tbl, tail0_emb, tail0_lin, _HEAD, _CUT1 - _HEAD, 64, 5000)
    tbl = proj(tbl, tail1_emb, tail1_lin, _CUT1, _VOCAB - _CUT1, 32, 25000)
    return tbl


def _make_gather(n_tokens):
    info = plsc.get_sparse_core_info()
    nc, ns = info.num_cores, info.num_subcores
    nw = nc * ns
    chunk = 128
    spc = 2
    slot_rows = chunk * spc
    nbuf = 3
    per_w = n_tokens // nw
    n_chunks = per_w // chunk
    n_slots = n_chunks // spc
    assert per_w % slot_rows == 0 and n_slots % nbuf == 1
    mesh = plsc.VectorSubcoreMesh(core_axis_name="c", subcore_axis_name="s")

    @functools.partial(
        pl.kernel, mesh=mesh,
        out_type=jax.ShapeDtypeStruct((n_tokens, _DIM), jnp.float32),
        scratch_types=[
            pltpu.VMEM((n_chunks, chunk), jnp.int32),
        ]
        + [pltpu.VMEM((slot_rows, _DIM), jnp.float32)] * nbuf
        + [pltpu.SemaphoreType.DMA] * (2 * nbuf),
    )
    def gather(table_hbm, idx_hbm, out_hbm, idx_v, *bufsem):
        bufs = bufsem[:nbuf]
        gsem = bufsem[nbuf:2 * nbuf]
        ssem = bufsem[2 * nbuf:]
        wid = lax.axis_index("s") * nc + lax.axis_index("c")
        row0 = wid * per_w
        pltpu.sync_copy(idx_hbm.at[pl.ds(wid * n_chunks, n_chunks)], idx_v)

        def start_slot(s, b):
            for c in range(spc):
                pltpu.async_copy(
                    table_hbm.at[idx_v.at[s * spc + c]],
                    bufs[b].at[pl.ds(c * chunk, chunk)], gsem[b])

        def wait_slot(s, b):
            for c in range(spc):
                pltpu.make_async_copy(
                    table_hbm.at[idx_v.at[s * spc + c]],
                    bufs[b].at[pl.ds(c * chunk, chunk)], gsem[b]).wait()

        def wait_store(b):
            pltpu.make_async_copy(
                bufs[b], out_hbm.at[pl.ds(row0, slot_rows)], ssem[b]).wait()

        for b in range(nbuf - 1):
            start_slot(b, b)

        def body(g, _):
            for b in range(nbuf):
                s = g * nbuf + b
                wait_slot(s, b)
                pltpu.async_copy(
                    bufs[b], out_hbm.at[pl.ds(row0 + s * slot_rows, slot_rows)],
                    ssem[b])
                sn = s + nbuf - 1
                bn = (b - 1) % nbuf

                @pl.when(sn < n_slots)
                def _():
                    @pl.when(s >= 1)
                    def _():
                        wait_store(bn)

                    start_slot(sn, bn)

        lax.fori_loop(0, n_slots // nbuf, body, None)
        s_last = n_slots - 1
        b_last = s_last % nbuf
        wait_slot(s_last, b_last)
        pltpu.async_copy(
            bufs[b_last],
            out_hbm.at[pl.ds(row0 + s_last * slot_rows, slot_rows)],
            ssem[b_last])
        for b in range(nbuf):
            wait_store(b)

    return gather


def kernel(input, head_w, tail0_emb, tail0_lin, tail1_emb, tail1_lin):
    B, L = input.shape
    n_tokens = B * L
    table = _build_table(head_w, tail0_emb, tail0_lin, tail1_emb, tail1_lin)
    idx2d = input.reshape(n_tokens // 128, 128)
    out = _make_gather(n_tokens)(table, idx2d)
    return out.reshape(B, L, _DIM)
